# baseline (speedup 1.0000x reference)
import functools

import jax
import jax.numpy as jnp
from jax import lax
from jax.experimental import pallas as pl
from jax.experimental.pallas import tpu as pltpu
from jax.experimental.pallas import tpu_sc as plsc

_B = 4096
_L = 200
_DIM = 64
_LP = 104
_NW = 32
_BPW = _B // _NW


def _make_kernel():
    mesh = plsc.VectorSubcoreMesh(core_axis_name="c", subcore_axis_name="s")

    @functools.partial(
        pl.kernel,
        mesh=mesh,
        out_type=jax.ShapeDtypeStruct((_B, _DIM), jnp.float32),
        compiler_params=pltpu.CompilerParams(use_tc_tiling_on_sc=False),
        scratch_types=[
            pltpu.VMEM((_BPW, 2, _LP), jnp.int32),
            pltpu.VMEM((2 * _LP, _DIM), jnp.float32),
            pltpu.VMEM((_BPW, _DIM), jnp.float32),
            pltpu.SemaphoreType.DMA,
        ],
    )
    def k(x_hbm, table_hbm, out_hbm, idx_v, rows_v, out_v, sem):
        wid = lax.axis_index("s") * 2 + lax.axis_index("c")
        base = wid * _BPW
        pltpu.sync_copy(x_hbm.at[pl.ds(base, _BPW)], idx_v)

        def row_body(b, carry):
            cp0 = pltpu.async_copy(
                table_hbm.at[idx_v.at[b, 0]], rows_v.at[pl.ds(0, _LP)], sem)
            cp1 = pltpu.async_copy(
                table_hbm.at[idx_v.at[b, 1]], rows_v.at[pl.ds(_LP, _LP)], sem)
            cp0.wait()
            cp1.wait()

            def acc_body(r, accs):
                return tuple(
                    a
                    + rows_v[r, pl.ds(16 * c, 16)]
                    + rows_v[_LP + r, pl.ds(16 * c, 16)]
                    for c, a in enumerate(accs)
                )

            accs = lax.fori_loop(
                0, _LP, acc_body,
                tuple(jnp.zeros((16,), jnp.float32) for _ in range(4)))
            for c in range(4):
                out_v[b, pl.ds(16 * c, 16)] = accs[c] * (1.0 / _L)
            return carry

        lax.fori_loop(0, _BPW, row_body, 0)
        pltpu.sync_copy(out_v, out_hbm.at[pl.ds(base, _BPW)])

    return k


_kernel_call = _make_kernel()


def kernel(X, table):
    Xp = jnp.pad(X.reshape(_B, 2, _L // 2), ((0, 0), (0, 0), (0, _LP - _L // 2)))
    return _kernel_call(Xp, table)
